# baseline (speedup 1.0000x reference)
import functools

import jax
import jax.numpy as jnp
from jax import lax
from jax.experimental import pallas as pl
from jax.experimental.pallas import tpu as pltpu
from jax.experimental.pallas import tpu_sc as plsc

_HIDDEN = 1024
_NC = 2
_NS = 16
_NW = _NC * _NS
_CHUNK = 40


@functools.cache
def _build(num_rows):
    assert num_rows % (_NW * _CHUNK) == 0
    rows_per_w = num_rows // _NW
    n_chunks = rows_per_w // _CHUNK
    mesh = plsc.VectorSubcoreMesh(core_axis_name="c", subcore_axis_name="s")

    @functools.partial(
        pl.kernel,
        mesh=mesh,
        out_type=jax.ShapeDtypeStruct((num_rows, _HIDDEN), jnp.float32),
        scratch_types=[
            pltpu.VMEM((rows_per_w,), jnp.int32),
            pltpu.VMEM((2, _CHUNK, _HIDDEN), jnp.float32),
            pltpu.SemaphoreType.DMA,
            pltpu.SemaphoreType.DMA,
        ],
    )
    def emb(idx_hbm, table_hbm, out_hbm, idx_v, rows_v, gsem, ssem):
        wid = lax.axis_index("s") * _NC + lax.axis_index("c")
        base = wid * rows_per_w
        pltpu.sync_copy(idx_hbm.at[pl.ds(base, rows_per_w)], idx_v)

        def gather(c, buf):
            return pltpu.async_copy(
                table_hbm.at[idx_v.at[pl.ds(c * _CHUNK, _CHUNK)]],
                rows_v.at[buf],
                gsem,
            )

        def put(c, buf):
            return pltpu.async_copy(
                rows_v.at[buf],
                out_hbm.at[pl.ds(base + c * _CHUNK, _CHUNK)],
                ssem,
            )

        gh = [None] * n_chunks
        sh = [None] * n_chunks
        gh[0] = gather(0, 0)
        for c in range(n_chunks):
            buf = c % 2
            if c + 1 < n_chunks:
                if c >= 1:
                    sh[c - 1].wait()
                gh[c + 1] = gather(c + 1, 1 - buf)
            gh[c].wait()
            sh[c] = put(c, buf)
        if n_chunks >= 2:
            sh[n_chunks - 2].wait()
        sh[n_chunks - 1].wait()

    return emb


def kernel(indices, table):
    b, s = indices.shape
    idx = indices.reshape(b * s).astype(jnp.int32)
    out = _build(b * s)(idx, table)
    return out.reshape(b, s, _HIDDEN)


# speedup vs baseline: 1.0085x; 1.0085x over previous
import functools

import jax
import jax.numpy as jnp
from jax import lax
from jax.experimental import pallas as pl
from jax.experimental.pallas import tpu as pltpu
from jax.experimental.pallas import tpu_sc as plsc

_HIDDEN = 1024
_NC = 2
_NS = 16
_NW = _NC * _NS
_CHUNK = 16
_D = 5


@functools.cache
def _build(num_rows):
    assert num_rows % (_NW * _CHUNK * _D) == 0
    rows_per_w = num_rows // _NW
    n_chunks = rows_per_w // _CHUNK
    n_groups = n_chunks // _D
    mesh = plsc.VectorSubcoreMesh(core_axis_name="c", subcore_axis_name="s")

    @functools.partial(
        pl.kernel,
        mesh=mesh,
        out_type=jax.ShapeDtypeStruct((num_rows, _HIDDEN), jnp.float32),
        scratch_types=[
            pltpu.VMEM((rows_per_w,), jnp.int32),
            pltpu.VMEM((_D, _CHUNK, _HIDDEN), jnp.float32),
            pltpu.SemaphoreType.DMA,
            pltpu.SemaphoreType.DMA,
        ],
    )
    def emb(idx_hbm, table_hbm, out_hbm, idx_v, rows_v, gsem, ssem):
        wid = lax.axis_index("s") * _NC + lax.axis_index("c")
        base = wid * rows_per_w
        pltpu.sync_copy(idx_hbm.at[pl.ds(base, rows_per_w)], idx_v)

        def gather(c, j):
            return pltpu.make_async_copy(
                table_hbm.at[idx_v.at[pl.ds(c * _CHUNK, _CHUNK)]],
                rows_v.at[j], gsem)

        def store(c, j):
            return pltpu.make_async_copy(
                rows_v.at[j],
                out_hbm.at[pl.ds(base + c * _CHUNK, _CHUNK)], ssem)

        for j in range(_D - 1):
            gather(j, j).start()

        for j in range(_D):
            if j >= 1:
                store(j - 1, j - 1).wait()
            gather(j + _D - 1, (j + _D - 1) % _D).start()
            gather(j, j).wait()
            store(j, j).start()

        def body(g, _):
            c0 = g * _D
            for j in range(_D):
                c = c0 + j
                store(c - 1, (j - 1) % _D).wait()
                gather(c + _D - 1, (j + _D - 1) % _D).start()
                gather(c, j).wait()
                store(c, j).start()
            return 0

        lax.fori_loop(1, n_groups - 1, body, 0)

        c0 = (n_groups - 1) * _D
        for j in range(_D):
            c = c0 + j
            store(c - 1, (j - 1) % _D).wait()
            if j == 0:
                gather(c0 + _D - 1, _D - 1).start()
            gather(c, j).wait()
            store(c, j).start()
        store(c0 + _D - 1, _D - 1).wait()

    return emb


def kernel(indices, table):
    b, s = indices.shape
    idx = indices.reshape(b * s).astype(jnp.int32)
    out = _build(b * s)(idx, table)
    return out.reshape(b, s, _HIDDEN)


# speedup vs baseline: 4.1033x; 4.0687x over previous
import functools

import jax
import jax.numpy as jnp
from jax import lax
from jax.experimental import pallas as pl
from jax.experimental.pallas import tpu as pltpu
from jax.experimental.pallas import tpu_sc as plsc

_HIDDEN = 1024
_NC = 2
_NS = 16
_NW = _NC * _NS
_GRP = 4


@functools.cache
def _build(batch, seq, vocab):
    assert batch % (_NW * _GRP) == 0 and (_GRP * seq) % 16 == 0
    bpw = batch // _NW
    mesh = plsc.VectorSubcoreMesh(core_axis_name="c", subcore_axis_name="s")

    @functools.partial(
        pl.kernel,
        mesh=mesh,
        out_type=jax.ShapeDtypeStruct((batch, seq, _HIDDEN), jnp.float32),
        scratch_types=[
            pltpu.VMEM((vocab, _HIDDEN), jnp.float32),
            pltpu.VMEM((bpw * seq,), jnp.int32),
            pltpu.SemaphoreType.DMA,
            pltpu.SemaphoreType.DMA,
        ],
    )
    def emb(idx_hbm, table_hbm, out_hbm, table_v, idx_v, tsem, ssem):
        wid = lax.axis_index("s") * _NC + lax.axis_index("c")
        bbase = wid * bpw
        pltpu.async_copy(table_hbm, table_v, tsem).wait()
        pltpu.sync_copy(idx_hbm.at[pl.ds(bbase * seq, bpw * seq)], idx_v)

        def store(b, sp, row):
            return pltpu.make_async_copy(
                table_v.at[pl.ds(row, 1)],
                out_hbm.at[b, pl.ds(sp, 1)],
                ssem,
            )

        npos = _GRP * seq
        nvec = npos // 16

        def fire_group(g):
            vs = [idx_v[pl.ds(g * npos + 16 * k, 16)] for k in range(nvec)]
            for p in range(npos):
                b = bbase + g * _GRP + p // seq
                store(b, p % seq, vs[p // 16][p % 16]).start()

        def wait_group():
            for _ in range(npos):
                store(bbase, 0, 0).wait()

        fire_group(0)

        def grp_body(g, _):
            wait_group()
            fire_group(g)
            return 0

        lax.fori_loop(1, bpw // _GRP, grp_body, 0)
        wait_group()

    return emb


def kernel(indices, table):
    b, s = indices.shape
    v = table.shape[0]
    idx = indices.reshape(b * s).astype(jnp.int32)
    return _build(b, s, v)(idx, table)
